# baseline (speedup 1.0000x reference)
import functools

import jax
import jax.numpy as jnp
from jax import lax
from jax.experimental import pallas as pl
from jax.experimental.pallas import tpu as pltpu
from jax.experimental.pallas import tpu_sc as plsc

B = 1024
S = 200
D = 64
N = B * S
V = 1000000
HV = V // 2
K = 499968
EPS = 1e-5


def _mesh():
    return plsc.VectorSubcoreMesh(core_axis_name="c", subcore_axis_name="s")


WROWS = V - K + 32
CB = 3968


def _tc_widen(tok_t):
    grid = (pl.cdiv(WROWS, CB),)

    def body(lo_ref, hi_ref, o_ref):
        eye = (lax.broadcasted_iota(jnp.int32, (D, D), 0)
               == lax.broadcasted_iota(jnp.int32, (D, D), 1)).astype(jnp.float32)

        def t(x):
            return lax.dot_general(
                x, eye, (((0,), (0,)), ((), ())),
                precision=lax.Precision.HIGHEST,
            )

        o_ref[...] = jnp.concatenate([t(lo_ref[...]), t(hi_ref[...])], axis=-1)

    return pl.pallas_call(
        body,
        grid=grid,
        in_specs=[
            pl.BlockSpec((D, CB), lambda i: (0, i)),
            pl.BlockSpec((D, CB), lambda i: (0, i + K // CB)),
        ],
        out_specs=pl.BlockSpec((CB, 2 * D), lambda i: (i, 0)),
        out_shape=jax.ShapeDtypeStruct((WROWS, 2 * D), jnp.float32),
    )(tok_t, tok_t)


def _sc_gather(tok_wide, idx):
    info = plsc.get_sparse_core_info()
    nc, ns = info.num_cores, info.num_subcores
    nw = nc * ns
    b_per_w = N // nw
    ch = 400
    n_ch = b_per_w // ch

    @functools.partial(
        pl.kernel,
        mesh=_mesh(),
        compiler_params=pltpu.CompilerParams(use_tc_tiling_on_sc=True),
        out_type=jax.ShapeDtypeStruct((N, 2 * D), jnp.float32),
        scratch_types=[
            pltpu.VMEM((ch,), jnp.int32),
            pltpu.VMEM((ch,), jnp.int32),
            pltpu.VMEM((ch, 2 * D), jnp.float32),
            pltpu.VMEM((ch, 2 * D), jnp.float32),
            pltpu.SemaphoreType.DMA,
            pltpu.SemaphoreType.DMA,
            pltpu.SemaphoreType.DMA,
        ],
    )
    def k(tok_hbm, idx_hbm, out_hbm, idx_v0, idx_v1, rows_v0, rows_v1,
          gsem0, gsem1, osem):
        wid = lax.axis_index("s") * nc + lax.axis_index("c")
        base = wid * b_per_w
        idx_bufs = (idx_v0, idx_v1)
        row_bufs = (rows_v0, rows_v1)
        gsems = (gsem0, gsem1)

        def issue(c):
            slot = c % 2
            off = base + c * ch
            pltpu.sync_copy(idx_hbm.at[pl.ds(off, ch)], idx_bufs[slot])
            pltpu.async_copy(tok_hbm.at[idx_bufs[slot]], row_bufs[slot], gsems[slot])

        issue(0)
        for c in range(n_ch):
            slot = c % 2
            if c + 1 < n_ch:
                issue(c + 1)
            pltpu.make_async_copy(
                tok_hbm.at[idx_bufs[slot]], row_bufs[slot], gsems[slot]
            ).wait()
            off = base + c * ch
            copy = pltpu.make_async_copy(
                row_bufs[slot], out_hbm.at[pl.ds(off, ch)], osem
            )
            copy.start()
            copy.wait()

    return k(tok_wide, idx)


def _tc_add_ln(rows, xt, segt, pos_e, seg_e, gamma, beta):
    sb = 8
    grid = (S // sb,)

    def body(rows_ref, x_ref, seg_ref, pe_ref, se_ref, g_ref, b_ref, o_ref):
        h2 = rows_ref[...]
        parf = (x_ref[...] >= K).astype(jnp.float32)[:, :, None]
        lo = h2[:, :, :D]
        hi = h2[:, :, D:]
        h = lo + parf * (hi - lo)
        se0 = se_ref[0, :]
        sed = se_ref[1, :] - se_ref[0, :]
        segf = seg_ref[...].astype(jnp.float32)
        h = h + pe_ref[...][:, None, :] + se0[None, None, :] \
            + segf[:, :, None] * sed[None, None, :]
        mean = jnp.mean(h, axis=-1, keepdims=True)
        hc = h - mean
        var = jnp.mean(hc * hc, axis=-1, keepdims=True)
        o_ref[...] = hc * lax.rsqrt(var + EPS) * g_ref[0, :][None, None, :] \
            + b_ref[0, :][None, None, :]

    return pl.pallas_call(
        body,
        grid=grid,
        in_specs=[
            pl.BlockSpec((sb, B, 2 * D), lambda i: (i, 0, 0)),
            pl.BlockSpec((sb, B), lambda i: (i, 0)),
            pl.BlockSpec((sb, B), lambda i: (i, 0)),
            pl.BlockSpec((sb, D), lambda i: (i, 0)),
            pl.BlockSpec((2, D), lambda i: (0, 0)),
            pl.BlockSpec((1, D), lambda i: (0, 0)),
            pl.BlockSpec((1, D), lambda i: (0, 0)),
        ],
        out_specs=pl.BlockSpec((sb, B, D), lambda i: (i, 0, 0)),
        out_shape=jax.ShapeDtypeStruct((S, B, D), jnp.float32),
    )(rows, xt, segt, pos_e, seg_e, gamma, beta)


def kernel(x, seg, tok_embed, pos_embed, seg_embed, gamma, beta):
    x = x.astype(jnp.int32)
    xt = jnp.swapaxes(x, 0, 1)
    xflat = xt.reshape(N)
    idx_mod = jnp.where(xflat >= K, xflat - K, xflat)
    tok_wide = _tc_widen(jnp.swapaxes(tok_embed, 0, 1))
    rows = _sc_gather(tok_wide, idx_mod)
    rows = rows.reshape(S, B, 2 * D)
    out_sbd = _tc_add_ln(
        rows,
        xt,
        jnp.swapaxes(seg.astype(jnp.int32), 0, 1),
        pos_embed[:S],
        seg_embed,
        gamma.reshape(1, D),
        beta.reshape(1, D),
    )
    return jnp.transpose(out_sbd, (1, 0, 2))


# speedup vs baseline: 1.6103x; 1.6103x over previous
import functools

import jax
import jax.numpy as jnp
from jax import lax
from jax.experimental import pallas as pl
from jax.experimental.pallas import tpu as pltpu
from jax.experimental.pallas import tpu_sc as plsc

B = 1024
S = 200
D = 64
N = B * S
V = 1000000
HV = V // 2
K = 499968
EPS = 1e-5


def _mesh():
    return plsc.VectorSubcoreMesh(core_axis_name="c", subcore_axis_name="s")


WROWS = V - K + 32
CB = 3968


def _tc_widen(tok_t):
    grid = (pl.cdiv(WROWS, CB),)

    def body(lo_ref, hi_ref, o_ref):
        in2 = jnp.concatenate([lo_ref[...], hi_ref[...]], axis=0)
        o_ref[...] = jnp.swapaxes(in2, 0, 1)

    return pl.pallas_call(
        body,
        grid=grid,
        in_specs=[
            pl.BlockSpec((D, CB), lambda i: (0, i)),
            pl.BlockSpec((D, CB), lambda i: (0, i + K // CB)),
        ],
        out_specs=pl.BlockSpec((CB, 2 * D), lambda i: (i, 0)),
        out_shape=jax.ShapeDtypeStruct((WROWS, 2 * D), jnp.float32),
    )(tok_t, tok_t)


def _sc_gather(tok_wide, idx):
    info = plsc.get_sparse_core_info()
    nc, ns = info.num_cores, info.num_subcores
    nw = nc * ns
    b_per_w = N // nw
    ch = 400
    n_ch = b_per_w // ch

    @functools.partial(
        pl.kernel,
        mesh=_mesh(),
        compiler_params=pltpu.CompilerParams(use_tc_tiling_on_sc=True),
        out_type=jax.ShapeDtypeStruct((N, 2 * D), jnp.float32),
        scratch_types=[
            pltpu.VMEM((ch,), jnp.int32),
            pltpu.VMEM((ch,), jnp.int32),
            pltpu.VMEM((ch, 2 * D), jnp.float32),
            pltpu.VMEM((ch, 2 * D), jnp.float32),
            pltpu.SemaphoreType.DMA,
            pltpu.SemaphoreType.DMA,
            pltpu.SemaphoreType.DMA,
        ],
    )
    def k(tok_hbm, idx_hbm, out_hbm, idx_v0, idx_v1, rows_v0, rows_v1,
          gsem0, gsem1, osem):
        wid = lax.axis_index("s") * nc + lax.axis_index("c")
        base = wid * b_per_w
        idx_bufs = (idx_v0, idx_v1)
        row_bufs = (rows_v0, rows_v1)
        gsems = (gsem0, gsem1)

        def issue(c):
            slot = c % 2
            off = base + c * ch
            pltpu.sync_copy(idx_hbm.at[pl.ds(off, ch)], idx_bufs[slot])
            pltpu.async_copy(tok_hbm.at[idx_bufs[slot]], row_bufs[slot], gsems[slot])

        issue(0)
        for c in range(n_ch):
            slot = c % 2
            if c + 1 < n_ch:
                issue(c + 1)
            pltpu.make_async_copy(
                tok_hbm.at[idx_bufs[slot]], row_bufs[slot], gsems[slot]
            ).wait()
            off = base + c * ch
            copy = pltpu.make_async_copy(
                row_bufs[slot], out_hbm.at[pl.ds(off, ch)], osem
            )
            copy.start()
            copy.wait()

    return k(tok_wide, idx)


def _tc_add_ln(rows, xt, segt, pos_e, seg_e, gamma, beta):
    sb = 8
    grid = (S // sb,)

    def body(rows_ref, x_ref, seg_ref, pe_ref, se_ref, g_ref, b_ref, o_ref):
        h2 = rows_ref[...]
        parf = (x_ref[...] >= K).astype(jnp.float32)[:, :, None]
        lo = h2[:, :, :D]
        hi = h2[:, :, D:]
        h = lo + parf * (hi - lo)
        se0 = se_ref[0, :]
        sed = se_ref[1, :] - se_ref[0, :]
        segf = seg_ref[...].astype(jnp.float32)
        h = h + pe_ref[...][:, None, :] + se0[None, None, :] \
            + segf[:, :, None] * sed[None, None, :]
        mean = jnp.mean(h, axis=-1, keepdims=True)
        hc = h - mean
        var = jnp.mean(hc * hc, axis=-1, keepdims=True)
        o_ref[...] = hc * lax.rsqrt(var + EPS) * g_ref[0, :][None, None, :] \
            + b_ref[0, :][None, None, :]

    return pl.pallas_call(
        body,
        grid=grid,
        in_specs=[
            pl.BlockSpec((sb, B, 2 * D), lambda i: (i, 0, 0)),
            pl.BlockSpec((sb, B), lambda i: (i, 0)),
            pl.BlockSpec((sb, B), lambda i: (i, 0)),
            pl.BlockSpec((sb, D), lambda i: (i, 0)),
            pl.BlockSpec((2, D), lambda i: (0, 0)),
            pl.BlockSpec((1, D), lambda i: (0, 0)),
            pl.BlockSpec((1, D), lambda i: (0, 0)),
        ],
        out_specs=pl.BlockSpec((sb, B, D), lambda i: (i, 0, 0)),
        out_shape=jax.ShapeDtypeStruct((S, B, D), jnp.float32),
    )(rows, xt, segt, pos_e, seg_e, gamma, beta)


def kernel(x, seg, tok_embed, pos_embed, seg_embed, gamma, beta):
    x = x.astype(jnp.int32)
    xt = jnp.swapaxes(x, 0, 1)
    xflat = xt.reshape(N)
    idx_mod = jnp.where(xflat >= K, xflat - K, xflat)
    tok_wide = _tc_widen(jnp.swapaxes(tok_embed, 0, 1))
    rows = _sc_gather(tok_wide, idx_mod)
    rows = rows.reshape(S, B, 2 * D)
    out_sbd = _tc_add_ln(
        rows,
        xt,
        jnp.swapaxes(seg.astype(jnp.int32), 0, 1),
        pos_embed[:S],
        seg_embed,
        gamma.reshape(1, D),
        beta.reshape(1, D),
    )
    return jnp.transpose(out_sbd, (1, 0, 2))


# speedup vs baseline: 1.7145x; 1.0647x over previous
import functools

import jax
import jax.numpy as jnp
from jax import lax
from jax.experimental import pallas as pl
from jax.experimental.pallas import tpu as pltpu
from jax.experimental.pallas import tpu_sc as plsc

B = 1024
S = 200
D = 64
N = B * S
V = 1000000
HV = V // 2
K = 499968
EPS = 1e-5


def _mesh():
    return plsc.VectorSubcoreMesh(core_axis_name="c", subcore_axis_name="s")


WROWS = V - K + 32
CB = 8064


def _tc_widen(tok_t):
    grid = (pl.cdiv(WROWS, CB),)

    def body(lo_ref, hi_ref, o_ref):
        in2 = jnp.concatenate([lo_ref[...], hi_ref[...]], axis=0)
        o_ref[...] = jnp.swapaxes(in2, 0, 1)

    return pl.pallas_call(
        body,
        grid=grid,
        in_specs=[
            pl.BlockSpec((D, CB), lambda i: (0, i)),
            pl.BlockSpec((D, CB), lambda i: (0, i + K // CB)),
        ],
        out_specs=pl.BlockSpec((CB, 2 * D), lambda i: (i, 0)),
        out_shape=jax.ShapeDtypeStruct((WROWS, 2 * D), jnp.float32),
    )(tok_t, tok_t)


def _sc_gather(tok_wide, idx):
    info = plsc.get_sparse_core_info()
    nc, ns = info.num_cores, info.num_subcores
    nw = nc * ns
    b_per_w = N // nw
    ch = 400
    n_ch = b_per_w // ch

    @functools.partial(
        pl.kernel,
        mesh=_mesh(),
        compiler_params=pltpu.CompilerParams(use_tc_tiling_on_sc=True),
        out_type=jax.ShapeDtypeStruct((N, 2 * D), jnp.float32),
        scratch_types=[
            pltpu.VMEM((ch,), jnp.int32),
            pltpu.VMEM((ch,), jnp.int32),
            pltpu.VMEM((ch, 2 * D), jnp.float32),
            pltpu.VMEM((ch, 2 * D), jnp.float32),
            pltpu.SemaphoreType.DMA,
            pltpu.SemaphoreType.DMA,
            pltpu.SemaphoreType.DMA,
        ],
    )
    def k(tok_hbm, idx_hbm, out_hbm, idx_v0, idx_v1, rows_v0, rows_v1,
          gsem0, gsem1, osem):
        wid = lax.axis_index("s") * nc + lax.axis_index("c")
        base = wid * b_per_w
        idx_bufs = (idx_v0, idx_v1)
        row_bufs = (rows_v0, rows_v1)
        gsems = (gsem0, gsem1)

        def issue(c):
            slot = c % 2
            off = base + c * ch
            pltpu.sync_copy(idx_hbm.at[pl.ds(off, ch)], idx_bufs[slot])
            pltpu.async_copy(tok_hbm.at[idx_bufs[slot]], row_bufs[slot], gsems[slot])

        issue(0)
        for c in range(n_ch):
            slot = c % 2
            if c + 1 < n_ch:
                issue(c + 1)
            pltpu.make_async_copy(
                tok_hbm.at[idx_bufs[slot]], row_bufs[slot], gsems[slot]
            ).wait()
            off = base + c * ch
            copy = pltpu.make_async_copy(
                row_bufs[slot], out_hbm.at[pl.ds(off, ch)], osem
            )
            copy.start()
            copy.wait()

    return k(tok_wide, idx)


def _tc_add_ln(rows, xt, segt, pos_e, seg_e, gamma, beta):
    sb = 8
    grid = (S // sb,)

    def body(rows_ref, x_ref, seg_ref, pe_ref, se_ref, g_ref, b_ref, o_ref):
        h2 = rows_ref[...]
        parf = (x_ref[...] >= K).astype(jnp.float32)[:, :, None]
        lo = h2[:, :, :D]
        hi = h2[:, :, D:]
        h = lo + parf * (hi - lo)
        se0 = se_ref[0, :]
        sed = se_ref[1, :] - se_ref[0, :]
        segf = seg_ref[...].astype(jnp.float32)
        h = h + pe_ref[...][:, None, :] + se0[None, None, :] \
            + segf[:, :, None] * sed[None, None, :]
        mean = jnp.mean(h, axis=-1, keepdims=True)
        hc = h - mean
        var = jnp.mean(hc * hc, axis=-1, keepdims=True)
        o_ref[...] = hc * lax.rsqrt(var + EPS) * g_ref[0, :][None, None, :] \
            + b_ref[0, :][None, None, :]

    return pl.pallas_call(
        body,
        grid=grid,
        in_specs=[
            pl.BlockSpec((sb, B, 2 * D), lambda i: (i, 0, 0)),
            pl.BlockSpec((sb, B), lambda i: (i, 0)),
            pl.BlockSpec((sb, B), lambda i: (i, 0)),
            pl.BlockSpec((sb, D), lambda i: (i, 0)),
            pl.BlockSpec((2, D), lambda i: (0, 0)),
            pl.BlockSpec((1, D), lambda i: (0, 0)),
            pl.BlockSpec((1, D), lambda i: (0, 0)),
        ],
        out_specs=pl.BlockSpec((sb, B, D), lambda i: (i, 0, 0)),
        out_shape=jax.ShapeDtypeStruct((S, B, D), jnp.float32),
    )(rows, xt, segt, pos_e, seg_e, gamma, beta)


def kernel(x, seg, tok_embed, pos_embed, seg_embed, gamma, beta):
    x = x.astype(jnp.int32)
    xt = jnp.swapaxes(x, 0, 1)
    xflat = xt.reshape(N)
    idx_mod = jnp.where(xflat >= K, xflat - K, xflat)
    tok_wide = _tc_widen(jnp.swapaxes(tok_embed, 0, 1))
    rows = _sc_gather(tok_wide, idx_mod)
    rows = rows.reshape(S, B, 2 * D)
    out_sbd = _tc_add_ln(
        rows,
        xt,
        jnp.swapaxes(seg.astype(jnp.int32), 0, 1),
        pos_embed[:S],
        seg_embed,
        gamma.reshape(1, D),
        beta.reshape(1, D),
    )
    return jnp.transpose(out_sbd, (1, 0, 2))
